# Initial kernel scaffold; baseline (speedup 1.0000x reference)
import functools

import jax
import jax.numpy as jnp
from jax import lax
from jax.experimental import pallas as pl
from jax.experimental.pallas import tpu as pltpu
from jax.experimental.pallas import tpu_sc as plsc

N = 100000
D = 128
S = 512
NC = 2
NS = 16
NW = NC * NS
R = N // NW
T = 256
G = D // 16

_mesh = plsc.VectorSubcoreMesh(core_axis_name="c", subcore_axis_name="s")


@functools.partial(
    pl.kernel,
    out_type=jax.ShapeDtypeStruct((S, D), jnp.float32),
    mesh=_mesh,
    scratch_types=[
        pltpu.VMEM((T + 8,), jnp.int32),
        pltpu.VMEM((T, D), jnp.float32),
        pltpu.VMEM((S, D), jnp.float32),
        pltpu.VMEM((16,), jnp.int32),
        pltpu.SemaphoreType.DMA,
    ],
)
def _pool_min_sc(feats_hbm, batch_hbm, out_hbm, ids_v, rows_v, outbuf, s16, sem):
    wid = lax.axis_index("s") * NC + lax.axis_index("c")
    c0 = wid * R
    c1 = c0 + R

    inf_vec = jnp.full((16,), jnp.inf, dtype=jnp.float32)

    pb = jnp.maximum((c0 - 1) // 8 * 8, 0)
    pltpu.sync_copy(batch_hbm.at[pl.ds(pb, 16)], s16)
    pid = jnp.where(wid == 0, jnp.int32(-1), s16[jnp.maximum(c0 - 1 - pb, 0)])

    lb = jnp.minimum((c1 - 1) // 8 * 8, N - 16)
    pltpu.sync_copy(batch_hbm.at[pl.ds(lb, 16)], s16)
    last_id = s16[c1 - 1 - lb]

    cover_hi = jnp.where(wid == NW - 1, jnp.int32(S - 1), last_id)
    count = cover_hi - pid

    def init_row(r, carry):
        for g in range(G):
            outbuf[r, pl.ds(g * 16, 16)] = inf_vec
        return carry

    lax.fori_loop(0, count, init_row, 0)

    def window_cond(carry):
        next_i, _, ext_go = carry[0], carry[1], carry[2]
        return (next_i < c1) | (ext_go & (next_i < N))

    def window_body(carry):
        next_i, cur_id0, _ = carry[0], carry[1], carry[2]
        accs0 = carry[3:]
        start_c = jnp.minimum(next_i, N - T)
        abase = jnp.minimum(start_c // 8 * 8, N - (T + 8))
        off = start_c - abase
        pltpu.sync_copy(batch_hbm.at[pl.ds(abase, T + 8)], ids_v)
        pltpu.sync_copy(feats_hbm.at[pl.ds(start_c, T)], rows_v)

        def row_body(j, rc):
            cur_id = rc[0]
            accs = rc[1:]
            gi = start_c + j
            vid = ids_v[off + j]
            active = (gi >= next_i) & (vid != pid) & ((gi < c1) | (vid == last_id))
            flush = active & (vid != cur_id) & (cur_id != pid)

            @pl.when(flush)
            def _():
                fr = cur_id - pid - 1
                for g in range(G):
                    outbuf[fr, pl.ds(g * 16, 16)] = accs[g]

            new_accs = []
            for g in range(G):
                a = jnp.where(flush, inf_vec, accs[g])
                row_g = rows_v[j, pl.ds(g * 16, 16)]
                new_accs.append(jnp.where(active, jnp.minimum(a, row_g), a))
            new_cur = jnp.where(active, vid, cur_id)
            return (new_cur,) + tuple(new_accs)

        rc = lax.fori_loop(0, T, row_body, (cur_id0,) + tuple(accs0))
        last_vid = ids_v[off + T - 1]
        next_i2 = start_c + T
        ext_go2 = (last_vid == last_id) & (pid != last_id)
        return (next_i2, rc[0], ext_go2) + tuple(rc[1:])

    init = (c0, pid, pid != last_id) + tuple([inf_vec] * G)
    fin = lax.while_loop(window_cond, window_body, init)
    cur_id = fin[1]
    accs = fin[3:]

    @pl.when(cur_id != pid)
    def _():
        fr = cur_id - pid - 1
        for g in range(G):
            outbuf[fr, pl.ds(g * 16, 16)] = accs[g]

    def fire(r, carry):
        pltpu.make_async_copy(outbuf.at[r], out_hbm.at[pid + 1 + r], sem).start()
        return carry

    lax.fori_loop(0, count, fire, 0)

    def drain(r, carry):
        pltpu.make_async_copy(outbuf.at[r], out_hbm.at[pid + 1 + r], sem).wait()
        return carry

    lax.fori_loop(0, count, drain, 0)


def kernel(feats, batch):
    return _pool_min_sc(feats, batch.astype(jnp.int32))


# speedup vs baseline: 4.3644x; 4.3644x over previous
import functools

import jax
import jax.numpy as jnp
from jax import lax
from jax.experimental import pallas as pl
from jax.experimental.pallas import tpu as pltpu
from jax.experimental.pallas import tpu_sc as plsc

N = 100000
D = 128
S = 512
NC = 2
NS = 16
NW = NC * NS
CH = 3136
T = 128
G = D // 16
GRP = T // 32
NPROBE = 8

_mesh = plsc.VectorSubcoreMesh(core_axis_name="c", subcore_axis_name="s")


@functools.partial(
    pl.kernel,
    out_type=jax.ShapeDtypeStruct((S, D), jnp.float32),
    mesh=_mesh,
    scratch_types=[
        pltpu.VMEM((T,), jnp.int32),
        pltpu.VMEM((T,), jnp.int32),
        pltpu.VMEM((T, D), jnp.float32),
        pltpu.VMEM((T, D), jnp.float32),
        pltpu.VMEM((S, D), jnp.float32),
        pltpu.VMEM((D,), jnp.float32),
        pltpu.VMEM((16,), jnp.int32),
        pltpu.SMEM((4,), jnp.int32),
        pltpu.SemaphoreType.DMA,
        pltpu.SemaphoreType.DMA,
        pltpu.SemaphoreType.DMA,
        pltpu.SemaphoreType.DMA,
        pltpu.SemaphoreType.DMA,
    ],
)
def _pool_min_sc(
    feats_hbm, batch_hbm, out_hbm, ids_a, ids_b, rows_a, rows_b, outbuf,
    acc_v, pbuf, st, sem_ia, sem_ib, sem_ra, sem_rb, sem
):
    wid = lax.axis_index("s") * NC + lax.axis_index("c")
    c0 = wid * CH
    c1 = jnp.minimum(c0 + CH, N)

    inf_vec = jnp.full((16,), jnp.inf, dtype=jnp.float32)

    pb = pl.multiple_of(jnp.maximum(c0 - 16, 0), 16)
    pltpu.sync_copy(batch_hbm.at[pl.ds(pb, 16)], pbuf)
    pid = jnp.where(wid == 0, jnp.int32(-1), pbuf[...][15])

    pltpu.sync_copy(batch_hbm.at[pl.ds(pl.multiple_of(c1 - 16, 16), 16)], pbuf)
    last_id = pbuf[...][15]

    cover_hi = jnp.where(wid == NW - 1, jnp.int32(S - 1), last_id)
    count = cover_hi - pid

    do_scan = pid != last_id

    @pl.when(do_scan)
    def _prime():
        s0 = pl.multiple_of(c0, 16)
        pltpu.make_async_copy(batch_hbm.at[pl.ds(s0, T)], ids_a, sem_ia).start()
        pltpu.make_async_copy(feats_hbm.at[pl.ds(s0, T)], rows_a, sem_ra).start()

    def init_row(r, carry):
        for g in range(G):
            outbuf[r, pl.ds(g * 16, 16)] = inf_vec
        return carry

    lax.fori_loop(0, count, init_row, 0)

    @pl.when(do_scan)
    def _scan():
        st[0] = jnp.int32(0)
        st[1] = jnp.int32(N)
        st[2] = pid
        for g in range(G):
            acc_v[pl.ds(g * 16, 16)] = inf_vec
        for m in range(NPROBE):
            @pl.when(st[0] == 0)
            def _probe(m=m):
                pos = pl.multiple_of(
                    jnp.minimum(c1 + (16 * 4**m - 16), N - 16), 16
                )
                pltpu.sync_copy(batch_hbm.at[pl.ds(pos, 16)], pbuf)
                f = pbuf[...][15] != last_id

                @pl.when(f | (pos >= N - 16))
                def _():
                    st[0] = jnp.int32(1)
                    st[1] = jnp.minimum(pos + 16, N)

        scan_end = st[1]
        nwin = (scan_end - c0 + T - 1) // T

        bufs = (
            (ids_a, rows_a, sem_ia, sem_ra),
            (ids_b, rows_b, sem_ib, sem_rb),
        )

        def pair_body(kk, carry):
            for b in range(2):
                k = kk * 2 + b
                ids_v, rows_v, sem_i, sem_r = bufs[b]
                ids_n, rows_n, sem_in_, sem_rn = bufs[1 - b]

                @pl.when(k < nwin)
                def _window(k=k, ids_v=ids_v, rows_v=rows_v, sem_i=sem_i,
                            sem_r=sem_r, ids_n=ids_n, rows_n=rows_n,
                            sem_in_=sem_in_, sem_rn=sem_rn):
                    wm = c0 + k * T
                    start_c = pl.multiple_of(jnp.minimum(wm, N - T), 16)
                    pltpu.make_async_copy(
                        batch_hbm.at[pl.ds(start_c, T)], ids_v, sem_i
                    ).wait()
                    pltpu.make_async_copy(
                        feats_hbm.at[pl.ds(start_c, T)], rows_v, sem_r
                    ).wait()

                    @pl.when(k + 1 < nwin)
                    def _prefetch():
                        s2 = pl.multiple_of(
                            jnp.minimum(c0 + (k + 1) * T, N - T), 16
                        )
                        pltpu.make_async_copy(
                            batch_hbm.at[pl.ds(s2, T)], ids_n, sem_in_
                        ).start()
                        pltpu.make_async_copy(
                            feats_hbm.at[pl.ds(s2, T)], rows_n, sem_rn
                        ).start()

                    def half_body(base, idvec, gi0):
                        g0 = idvec[0]
                        g15 = idvec[15]
                        cur0 = st[2]
                        all_active = (
                            (gi0 >= wm)
                            & (g0 != pid)
                            & ((gi0 + 15 < c1) | (g15 == last_id))
                        )
                        fast = all_active & (g0 == g15) & (g0 == cur0)

                        @pl.when(fast)
                        def _fast():
                            for g in range(G):
                                a = acc_v[pl.ds(g * 16, 16)]
                                for j in range(16):
                                    a = jnp.minimum(
                                        a, rows_v[base + j, pl.ds(g * 16, 16)]
                                    )
                                acc_v[pl.ds(g * 16, 16)] = a

                        @pl.when(jnp.logical_not(fast))
                        def _slow():
                            cur_id = cur0
                            accs = [
                                acc_v[pl.ds(g * 16, 16)] for g in range(G)
                            ]
                            for j in range(16):
                                vid = idvec[j]
                                gi = gi0 + j
                                active = (
                                    (gi >= wm)
                                    & (vid != pid)
                                    & ((gi < c1) | (vid == last_id))
                                )
                                flush = (
                                    active
                                    & (vid != cur_id)
                                    & (cur_id != pid)
                                )

                                @pl.when(flush)
                                def _(cur_id=cur_id, snap=tuple(accs)):
                                    fr = cur_id - pid - 1
                                    for g in range(G):
                                        outbuf[fr, pl.ds(g * 16, 16)] = snap[g]

                                for g in range(G):
                                    a = jnp.where(flush, inf_vec, accs[g])
                                    accs[g] = jnp.where(
                                        active,
                                        jnp.minimum(
                                            a,
                                            rows_v[base + j, pl.ds(g * 16, 16)],
                                        ),
                                        a,
                                    )
                                cur_id = jnp.where(active, vid, cur_id)
                            for g in range(G):
                                acc_v[pl.ds(g * 16, 16)] = accs[g]
                            st[2] = cur_id

                    def grp_body(k2, carry2):
                        base = pl.multiple_of(k2 * 32, 16)
                        idvec = ids_v[pl.ds(base, 16)]
                        idvec2 = ids_v[pl.ds(base + 16, 16)]
                        g0 = idvec[0]
                        h15 = idvec2[15]
                        gi0 = start_c + base
                        act32 = (
                            (gi0 >= wm)
                            & (g0 != pid)
                            & ((gi0 + 31 < c1) | (h15 == last_id))
                        )
                        fast32 = act32 & (g0 == h15) & (g0 == st[2])

                        @pl.when(fast32)
                        def _fast32():
                            for g in range(G):
                                a = acc_v[pl.ds(g * 16, 16)]
                                for j in range(32):
                                    a = jnp.minimum(
                                        a, rows_v[base + j, pl.ds(g * 16, 16)]
                                    )
                                acc_v[pl.ds(g * 16, 16)] = a

                        @pl.when(jnp.logical_not(fast32))
                        def _halves():
                            half_body(base, idvec, gi0)
                            half_body(base + 16, idvec2, gi0 + 16)

                        return carry2

                    lax.fori_loop(0, GRP, grp_body, 0)

            return carry

        lax.fori_loop(0, (nwin + 1) // 2, pair_body, 0)

        cur_id = st[2]

        @pl.when(cur_id != pid)
        def _():
            fr = cur_id - pid - 1
            for g in range(G):
                outbuf[fr, pl.ds(g * 16, 16)] = acc_v[pl.ds(g * 16, 16)]

    def fire(r, carry):
        pltpu.make_async_copy(outbuf.at[r], out_hbm.at[pid + 1 + r], sem).start()
        return carry

    lax.fori_loop(0, count, fire, 0)

    def drain(r, carry):
        pltpu.make_async_copy(outbuf.at[r], out_hbm.at[pid + 1 + r], sem).wait()
        return carry

    lax.fori_loop(0, count, drain, 0)


def kernel(feats, batch):
    return _pool_min_sc(feats, batch.astype(jnp.int32))


# speedup vs baseline: 6.3042x; 1.4445x over previous
import functools

import jax
import jax.numpy as jnp
from jax import lax
from jax.experimental import pallas as pl
from jax.experimental.pallas import tpu as pltpu
from jax.experimental.pallas import tpu_sc as plsc

N = 100000
D = 128
S = 512
NC = 2
NS = 16
NW = NC * NS
CH = 3136
T = 128
G = D // 16
GRP = T // 16
NPROBE = 8

_mesh = plsc.VectorSubcoreMesh(core_axis_name="c", subcore_axis_name="s")


@functools.partial(
    pl.kernel,
    out_type=jax.ShapeDtypeStruct((S, D), jnp.float32),
    mesh=_mesh,
    scratch_types=[
        pltpu.VMEM((T,), jnp.int32),
        pltpu.VMEM((T,), jnp.int32),
        pltpu.VMEM((T, D), jnp.float32),
        pltpu.VMEM((T, D), jnp.float32),
        pltpu.VMEM((S, D), jnp.float32),
        pltpu.VMEM((D,), jnp.float32),
        pltpu.VMEM((16,), jnp.int32),
        pltpu.VMEM((16,), jnp.int32),
        pltpu.SMEM((4,), jnp.int32),
        pltpu.SemaphoreType.DMA,
        pltpu.SemaphoreType.DMA,
        pltpu.SemaphoreType.DMA,
        pltpu.SemaphoreType.DMA,
        pltpu.SemaphoreType.DMA,
    ],
)
def _pool_min_sc(
    feats_hbm, batch_hbm, out_hbm, ids_a, ids_b, rows_a, rows_b, outbuf,
    acc_v, pbuf, lbuf, st, sem_ia, sem_ib, sem_ra, sem_rb, sem
):
    wid = lax.axis_index("s") * NC + lax.axis_index("c")
    c0 = wid * CH
    c1 = jnp.minimum(c0 + CH, N)

    inf_vec = jnp.full((16,), jnp.inf, dtype=jnp.float32)

    pb = pl.multiple_of(jnp.maximum(c0 - 16, 0), 16)
    pid_cp = pltpu.make_async_copy(batch_hbm.at[pl.ds(pb, 16)], pbuf, sem_ib)
    pid_cp.start()
    lb = pl.multiple_of(c1 - 16, 16)
    last_cp = pltpu.make_async_copy(batch_hbm.at[pl.ds(lb, 16)], lbuf, sem_rb)
    last_cp.start()
    s0 = pl.multiple_of(c0, 16)
    pltpu.make_async_copy(batch_hbm.at[pl.ds(s0, T)], ids_a, sem_ia).start()
    pltpu.make_async_copy(feats_hbm.at[pl.ds(s0, T)], rows_a, sem_ra).start()

    pid_cp.wait()
    last_cp.wait()
    pid = jnp.where(wid == 0, jnp.int32(-1), pbuf[...][15])
    last_id = lbuf[...][15]

    cover_hi = jnp.where(wid == NW - 1, jnp.int32(S - 1), last_id)
    count = cover_hi - pid

    do_scan = pid != last_id

    @pl.when(jnp.logical_not(do_scan))
    def _drain_prime():
        pltpu.make_async_copy(batch_hbm.at[pl.ds(s0, T)], ids_a, sem_ia).wait()
        pltpu.make_async_copy(feats_hbm.at[pl.ds(s0, T)], rows_a, sem_ra).wait()

    def init_row(r, carry):
        for g in range(G):
            outbuf[r, pl.ds(g * 16, 16)] = inf_vec
        return carry

    lax.fori_loop(0, count, init_row, 0)

    @pl.when(do_scan)
    def _scan():
        st[0] = jnp.int32(0)
        st[1] = jnp.int32(N)
        st[2] = pid
        for g in range(G):
            acc_v[pl.ds(g * 16, 16)] = inf_vec
        for m in range(NPROBE):
            @pl.when(st[0] == 0)
            def _probe(m=m):
                pos = pl.multiple_of(
                    jnp.minimum(c1 + (16 * 4**m - 16), N - 16), 16
                )
                pltpu.sync_copy(batch_hbm.at[pl.ds(pos, 16)], pbuf)
                f = pbuf[...][15] != last_id

                @pl.when(f | (pos >= N - 16))
                def _():
                    st[0] = jnp.int32(1)
                    st[1] = jnp.minimum(pos + 16, N)

        scan_end = st[1]
        nwin = (scan_end - c0 + T - 1) // T

        bufs = (
            (ids_a, rows_a, sem_ia, sem_ra),
            (ids_b, rows_b, sem_ib, sem_rb),
        )

        def pair_body(kk, carry):
            for b in range(2):
                k = kk * 2 + b
                ids_v, rows_v, sem_i, sem_r = bufs[b]
                ids_n, rows_n, sem_in_, sem_rn = bufs[1 - b]

                @pl.when(k < nwin)
                def _window(k=k, ids_v=ids_v, rows_v=rows_v, sem_i=sem_i,
                            sem_r=sem_r, ids_n=ids_n, rows_n=rows_n,
                            sem_in_=sem_in_, sem_rn=sem_rn):
                    wm = c0 + k * T
                    start_c = pl.multiple_of(jnp.minimum(wm, N - T), 16)
                    pltpu.make_async_copy(
                        batch_hbm.at[pl.ds(start_c, T)], ids_v, sem_i
                    ).wait()
                    pltpu.make_async_copy(
                        feats_hbm.at[pl.ds(start_c, T)], rows_v, sem_r
                    ).wait()

                    @pl.when(k + 1 < nwin)
                    def _prefetch():
                        s2 = pl.multiple_of(
                            jnp.minimum(c0 + (k + 1) * T, N - T), 16
                        )
                        pltpu.make_async_copy(
                            batch_hbm.at[pl.ds(s2, T)], ids_n, sem_in_
                        ).start()
                        pltpu.make_async_copy(
                            feats_hbm.at[pl.ds(s2, T)], rows_n, sem_rn
                        ).start()

                    def grp_body(k2, carry2):
                        base = pl.multiple_of(k2 * 16, 16)
                        idvec = ids_v[pl.ds(base, 16)]
                        g0 = idvec[0]
                        g15 = idvec[15]
                        gi0 = start_c + base
                        cur0 = st[2]
                        all_active = (
                            (gi0 >= wm)
                            & (g0 != pid)
                            & ((gi0 + 15 < c1) | (g15 == last_id))
                        )
                        fast = all_active & (g0 == g15) & (g0 == cur0)

                        @pl.when(fast)
                        def _fast():
                            for g in range(G):
                                a = acc_v[pl.ds(g * 16, 16)]
                                for j in range(16):
                                    a = jnp.minimum(
                                        a, rows_v[base + j, pl.ds(g * 16, 16)]
                                    )
                                acc_v[pl.ds(g * 16, 16)] = a

                        @pl.when(jnp.logical_not(fast))
                        def _slow():
                            cur_id = cur0
                            accs = [
                                acc_v[pl.ds(g * 16, 16)] for g in range(G)
                            ]
                            for j in range(16):
                                vid = idvec[j]
                                gi = gi0 + j
                                active = (
                                    (gi >= wm)
                                    & (vid != pid)
                                    & ((gi < c1) | (vid == last_id))
                                )
                                flush = (
                                    active
                                    & (vid != cur_id)
                                    & (cur_id != pid)
                                )

                                @pl.when(flush)
                                def _(cur_id=cur_id, snap=tuple(accs)):
                                    fr = cur_id - pid - 1
                                    for g in range(G):
                                        outbuf[fr, pl.ds(g * 16, 16)] = snap[g]

                                for g in range(G):
                                    a = jnp.where(flush, inf_vec, accs[g])
                                    accs[g] = jnp.where(
                                        active,
                                        jnp.minimum(
                                            a,
                                            rows_v[base + j, pl.ds(g * 16, 16)],
                                        ),
                                        a,
                                    )
                                cur_id = jnp.where(active, vid, cur_id)
                            for g in range(G):
                                acc_v[pl.ds(g * 16, 16)] = accs[g]
                            st[2] = cur_id

                        return carry2

                    lax.fori_loop(0, GRP, grp_body, 0)

            return carry

        lax.fori_loop(0, (nwin + 1) // 2, pair_body, 0)

        cur_id = st[2]

        @pl.when(cur_id != pid)
        def _():
            fr = cur_id - pid - 1
            for g in range(G):
                outbuf[fr, pl.ds(g * 16, 16)] = acc_v[pl.ds(g * 16, 16)]

    def fire(r, carry):
        pltpu.make_async_copy(outbuf.at[r], out_hbm.at[pid + 1 + r], sem).start()
        return carry

    lax.fori_loop(0, count, fire, 0)

    def drain(r, carry):
        pltpu.make_async_copy(outbuf.at[r], out_hbm.at[pid + 1 + r], sem).wait()
        return carry

    lax.fori_loop(0, count, drain, 0)


def kernel(feats, batch):
    return _pool_min_sc(feats, batch.astype(jnp.int32))


# speedup vs baseline: 6.7431x; 1.0696x over previous
import functools

import jax
import jax.numpy as jnp
from jax import lax
from jax.experimental import pallas as pl
from jax.experimental.pallas import tpu as pltpu
from jax.experimental.pallas import tpu_sc as plsc

N = 100000
D = 128
S = 512
NC = 2
NS = 16
NW = NC * NS
CH = 3136
T = 128
G = D // 16
GRP = T // 16
NPROBE = 8

_mesh = plsc.VectorSubcoreMesh(core_axis_name="c", subcore_axis_name="s")


@functools.partial(
    pl.kernel,
    out_type=jax.ShapeDtypeStruct((S, D), jnp.float32),
    mesh=_mesh,
    scratch_types=[
        pltpu.VMEM((T,), jnp.int32),
        pltpu.VMEM((T,), jnp.int32),
        pltpu.VMEM((T,), jnp.int32),
        pltpu.VMEM((T, D), jnp.float32),
        pltpu.VMEM((T, D), jnp.float32),
        pltpu.VMEM((T, D), jnp.float32),
        pltpu.VMEM((S, D), jnp.float32),
        pltpu.VMEM((D,), jnp.float32),
        pltpu.VMEM((16,), jnp.int32),
        pltpu.VMEM((16,), jnp.int32),
        pltpu.SMEM((4,), jnp.int32),
        pltpu.SemaphoreType.DMA,
        pltpu.SemaphoreType.DMA,
        pltpu.SemaphoreType.DMA,
        pltpu.SemaphoreType.DMA,
        pltpu.SemaphoreType.DMA,
        pltpu.SemaphoreType.DMA,
        pltpu.SemaphoreType.DMA,
    ],
)
def _pool_min_sc(
    feats_hbm, batch_hbm, out_hbm, ids_a, ids_b, ids_c, rows_a, rows_b,
    rows_c, outbuf, acc_v, pbuf, lbuf, st, sem_ia, sem_ib, sem_ic,
    sem_ra, sem_rb, sem_rc, sem
):
    wid = lax.axis_index("s") * NC + lax.axis_index("c")
    c0 = wid * CH
    c1 = jnp.minimum(c0 + CH, N)

    inf_vec = jnp.full((16,), jnp.inf, dtype=jnp.float32)

    pb = pl.multiple_of(jnp.maximum(c0 - 16, 0), 16)
    pid_cp = pltpu.make_async_copy(batch_hbm.at[pl.ds(pb, 16)], pbuf, sem_ib)
    pid_cp.start()
    lb = pl.multiple_of(c1 - 16, 16)
    last_cp = pltpu.make_async_copy(batch_hbm.at[pl.ds(lb, 16)], lbuf, sem_rb)
    last_cp.start()
    s0 = pl.multiple_of(c0, 16)
    pltpu.make_async_copy(batch_hbm.at[pl.ds(s0, T)], ids_a, sem_ia).start()
    pltpu.make_async_copy(feats_hbm.at[pl.ds(s0, T)], rows_a, sem_ra).start()

    pid_cp.wait()
    last_cp.wait()
    pid = jnp.where(wid == 0, jnp.int32(-1), pbuf[...][15])
    last_id = lbuf[...][15]

    cover_hi = jnp.where(wid == NW - 1, jnp.int32(S - 1), last_id)
    count = cover_hi - pid

    do_scan = pid != last_id

    @pl.when(jnp.logical_not(do_scan))
    def _drain_prime():
        pltpu.make_async_copy(batch_hbm.at[pl.ds(s0, T)], ids_a, sem_ia).wait()
        pltpu.make_async_copy(feats_hbm.at[pl.ds(s0, T)], rows_a, sem_ra).wait()

    def init_row(r, carry):
        for g in range(G):
            outbuf[r, pl.ds(g * 16, 16)] = inf_vec
        return carry

    lax.fori_loop(0, count, init_row, 0)

    @pl.when(do_scan)
    def _scan():
        st[0] = jnp.int32(0)
        st[1] = jnp.int32(N)
        st[2] = pid
        for g in range(G):
            acc_v[pl.ds(g * 16, 16)] = inf_vec
        for m in range(NPROBE):
            @pl.when(st[0] == 0)
            def _probe(m=m):
                pos = pl.multiple_of(
                    jnp.minimum(c1 + (16 * 4**m - 16), N - 16), 16
                )
                pltpu.sync_copy(batch_hbm.at[pl.ds(pos, 16)], pbuf)
                f = pbuf[...][15] != last_id

                @pl.when(f | (pos >= N - 16))
                def _():
                    st[0] = jnp.int32(1)
                    st[1] = jnp.minimum(pos + 16, N)

        scan_end = st[1]
        nwin = (scan_end - c0 + T - 1) // T

        bufs = (
            (ids_a, rows_a, sem_ia, sem_ra),
            (ids_b, rows_b, sem_ib, sem_rb),
            (ids_c, rows_c, sem_ic, sem_rc),
        )

        @pl.when(nwin > 1)
        def _prime1():
            s1 = pl.multiple_of(jnp.minimum(c0 + T, N - T), 16)
            pltpu.make_async_copy(batch_hbm.at[pl.ds(s1, T)], ids_b, sem_ib).start()
            pltpu.make_async_copy(feats_hbm.at[pl.ds(s1, T)], rows_b, sem_rb).start()

        def trip_body(kk, carry):
            for b in range(3):
                k = kk * 3 + b
                ids_v, rows_v, sem_i, sem_r = bufs[b]
                ids_n, rows_n, sem_in_, sem_rn = bufs[(b + 2) % 3]

                @pl.when(k < nwin)
                def _window(k=k, ids_v=ids_v, rows_v=rows_v, sem_i=sem_i,
                            sem_r=sem_r, ids_n=ids_n, rows_n=rows_n,
                            sem_in_=sem_in_, sem_rn=sem_rn):
                    wm = c0 + k * T
                    start_c = pl.multiple_of(jnp.minimum(wm, N - T), 16)
                    pltpu.make_async_copy(
                        batch_hbm.at[pl.ds(start_c, T)], ids_v, sem_i
                    ).wait()
                    pltpu.make_async_copy(
                        feats_hbm.at[pl.ds(start_c, T)], rows_v, sem_r
                    ).wait()

                    @pl.when(k + 2 < nwin)
                    def _prefetch():
                        s2 = pl.multiple_of(
                            jnp.minimum(c0 + (k + 2) * T, N - T), 16
                        )
                        pltpu.make_async_copy(
                            batch_hbm.at[pl.ds(s2, T)], ids_n, sem_in_
                        ).start()
                        pltpu.make_async_copy(
                            feats_hbm.at[pl.ds(s2, T)], rows_n, sem_rn
                        ).start()

                    def grp_body(k2, carry2):
                        base = pl.multiple_of(k2 * 16, 16)
                        idvec = ids_v[pl.ds(base, 16)]
                        g0 = idvec[0]
                        g15 = idvec[15]
                        gi0 = start_c + base
                        cur0 = st[2]
                        all_active = (
                            (gi0 >= wm)
                            & (g0 != pid)
                            & ((gi0 + 15 < c1) | (g15 == last_id))
                        )
                        fast = all_active & (g0 == g15) & (g0 == cur0)

                        @pl.when(fast)
                        def _fast():
                            for g in range(G):
                                a = acc_v[pl.ds(g * 16, 16)]
                                for j in range(16):
                                    a = jnp.minimum(
                                        a, rows_v[base + j, pl.ds(g * 16, 16)]
                                    )
                                acc_v[pl.ds(g * 16, 16)] = a

                        @pl.when(jnp.logical_not(fast))
                        def _slow():
                            cur_id = cur0
                            accs = [
                                acc_v[pl.ds(g * 16, 16)] for g in range(G)
                            ]
                            for j in range(16):
                                vid = idvec[j]
                                gi = gi0 + j
                                active = (
                                    (gi >= wm)
                                    & (vid != pid)
                                    & ((gi < c1) | (vid == last_id))
                                )
                                flush = (
                                    active
                                    & (vid != cur_id)
                                    & (cur_id != pid)
                                )

                                @pl.when(flush)
                                def _(cur_id=cur_id, snap=tuple(accs)):
                                    fr = cur_id - pid - 1
                                    for g in range(G):
                                        outbuf[fr, pl.ds(g * 16, 16)] = snap[g]

                                for g in range(G):
                                    a = jnp.where(flush, inf_vec, accs[g])
                                    accs[g] = jnp.where(
                                        active,
                                        jnp.minimum(
                                            a,
                                            rows_v[base + j, pl.ds(g * 16, 16)],
                                        ),
                                        a,
                                    )
                                cur_id = jnp.where(active, vid, cur_id)
                            for g in range(G):
                                acc_v[pl.ds(g * 16, 16)] = accs[g]
                            st[2] = cur_id

                        return carry2

                    lax.fori_loop(0, GRP, grp_body, 0)

            return carry

        lax.fori_loop(0, (nwin + 2) // 3, trip_body, 0)

        cur_id = st[2]

        @pl.when(cur_id != pid)
        def _():
            fr = cur_id - pid - 1
            for g in range(G):
                outbuf[fr, pl.ds(g * 16, 16)] = acc_v[pl.ds(g * 16, 16)]

    def fire(r, carry):
        pltpu.make_async_copy(outbuf.at[r], out_hbm.at[pid + 1 + r], sem).start()
        return carry

    lax.fori_loop(0, count, fire, 0)

    def drain(r, carry):
        pltpu.make_async_copy(outbuf.at[r], out_hbm.at[pid + 1 + r], sem).wait()
        return carry

    lax.fori_loop(0, count, drain, 0)


def kernel(feats, batch):
    return _pool_min_sc(feats, batch.astype(jnp.int32))
